# baseline (speedup 1.0000x reference)
import jax
import jax.numpy as jnp
from jax.experimental import pallas as pl
from jax.experimental.pallas import tpu as pltpu

_SIGMA = 0.5
_INV = 1.0 / (_SIGMA * (2.0 ** 0.5))
_NPASS = 8
_NSTEP = 2
_R = 256


def _body(x_ref, mu_ref, nz_ref, out_ref, l0_ref, gv_scr, acc_s):
    p = pl.program_id(0)
    ph = pl.program_id(1)
    s = pl.program_id(2)

    @pl.when((p == 0) & (ph == 0) & (s == 0))
    def _init():
        acc_s[0] = 0.0

    @pl.when(ph == 0)
    def _phase_a():
        nz = nz_ref[:, 0, 0]
        mu = mu_ref[0, pl.ds(s * _R, _R), :]
        gv_scr[:, pl.ds(s * _R, _R), :] = jnp.clip(
            mu[None, :, :] + _SIGMA * nz, 0.0, 1.0)

        @pl.when(s == 0)
        def _erf():
            prob = 0.5 * (1.0 + jax.lax.erf(mu_ref[0] * _INV))
            acc_s[0] += jnp.sum(prob)

    @pl.when(ph == 1)
    def _phase_b():
        out_ref[:, 0, pl.ds(s * _R, _R), :] = (
            x_ref[:, 0, 0] * gv_scr[:, pl.ds(s * _R, _R), :])

    @pl.when((p == _NPASS - 1) & (ph == 1) & (s == _NSTEP - 1))
    def _final():
        l0_ref[...] = jnp.broadcast_to(acc_s[0], (1, 128))


@jax.jit
def kernel(input_tensor, mu, noise):
    b = input_tensor.shape[0]
    rows_per_pass = _NSTEP * _R
    x5 = input_tensor.reshape(b, _NPASS, _NSTEP, _R, 1024)
    nz5 = noise.reshape(b, _NPASS, _NSTEP, _R, 1024)
    mu3 = mu.reshape(_NPASS, rows_per_pass, 1024)
    gated, l0 = pl.pallas_call(
        _body,
        grid=(_NPASS, 2, _NSTEP),
        in_specs=[
            pl.BlockSpec((b, 1, 1, _R, 1024),
                         lambda p, ph, s: (0, p, jnp.where(ph == 1, s, 0), 0, 0)),
            pl.BlockSpec((1, rows_per_pass, 1024),
                         lambda p, ph, s: (p, 0, 0)),
            pl.BlockSpec((b, 1, 1, _R, 1024),
                         lambda p, ph, s: (0, p, jnp.where(ph == 0, s, _NSTEP - 1), 0, 0)),
        ],
        out_specs=[
            pl.BlockSpec((b, 1, rows_per_pass, 1024),
                         lambda p, ph, s: (0, p, 0, 0)),
            pl.BlockSpec((1, 128), lambda p, ph, s: (0, 0)),
        ],
        out_shape=[
            jax.ShapeDtypeStruct((b, _NPASS, rows_per_pass, 1024), jnp.float32),
            jax.ShapeDtypeStruct((1, 128), jnp.float32),
        ],
        scratch_shapes=[
            pltpu.VMEM((b, rows_per_pass, 1024), jnp.float32),
            pltpu.SMEM((1,), jnp.float32),
        ],
    )(x5, mu3, nz5)
    return gated.reshape(input_tensor.shape), l0[0, 0]
